# baseline (speedup 1.0000x reference)
import functools

import jax
import jax.numpy as jnp
from jax import lax
from jax.experimental import pallas as pl
from jax.experimental.pallas import tpu as pltpu
from jax.experimental.pallas import tpu_sc as plsc

L16 = 16
PADW = 128


def _make_sc_kernel(B, L, D, V):
    info = plsc.get_sparse_core_info()
    NC, NS = info.num_cores, info.num_subcores
    NW = NC * NS
    BBLK = B // NW
    assert BBLK == 128 and L % 2 == 0 and D % L16 == 0
    NBV = BBLK // L16
    CG = 8

    mesh = plsc.VectorSubcoreMesh(core_axis_name="c", subcore_axis_name="s")

    @functools.partial(
        pl.kernel,
        mesh=mesh,
        compiler_params=pltpu.CompilerParams(
            use_tc_tiling_on_sc=True, needs_layout_passes=False
        ),
        out_type=jax.ShapeDtypeStruct((L, D, B), jnp.float32),
        scratch_types=[
            pltpu.VMEM((L, PADW), jnp.float32),
            pltpu.VMEM((L, BBLK), jnp.int32),
            pltpu.VMEM((BBLK, PADW), jnp.float32),
            pltpu.VMEM((BBLK, PADW), jnp.float32),
            pltpu.VMEM((D, BBLK), jnp.float32),
            pltpu.VMEM((D, BBLK), jnp.float32),
            pltpu.SemaphoreType.DMA,
            pltpu.SemaphoreType.DMA,
            pltpu.SemaphoreType.DMA,
            pltpu.SemaphoreType.DMA,
        ],
    )
    def sc_kernel(idx_hbm, word_hbm, pos_hbm, out_hbm,
                  pos_v, idx_v, rows0, rows1, t0, t1,
                  gsem0, gsem1, osem0, osem1):
        rows = (rows0, rows1)
        tb = (t0, t1)
        gsem = (gsem0, gsem1)
        osem = (osem0, osem1)
        wid = lax.axis_index("s") * NC + lax.axis_index("c")
        b0 = wid * BBLK
        pltpu.sync_copy(pos_hbm, pos_v)
        pltpu.sync_copy(idx_hbm.at[:, pl.ds(b0, BBLK)], idx_v)

        def issue_gather(l, p):
            pltpu.async_copy(word_hbm.at[idx_v.at[l]], rows[p], gsem[p])

        def wait_gather(p):
            pltpu.make_async_copy(
                word_hbm.at[pl.ds(0, BBLK)], rows[p], gsem[p]
            ).wait()

        def wait_writeback(p):
            pltpu.make_async_copy(
                tb[p], out_hbm.at[0, :, pl.ds(0, BBLK)], osem[p]
            ).wait()

        def transpose_add_flush(l, p):
            rowidx = [
                jnp.broadcast_to(jnp.int32(k * L16), (L16,))
                + lax.iota(jnp.int32, L16)
                for k in range(NBV)
            ]
            pidx = jnp.broadcast_to(l, (L16,))
            for cg in range(D // CG):
                pb = []
                for cc in range(CG):
                    c = cg * CG + cc
                    pb.append(
                        plsc.load_gather(
                            pos_v,
                            [pidx, jnp.broadcast_to(jnp.int32(c), (L16,))],
                        )
                    )
                for k in range(NBV):
                    for cc in range(CG):
                        c = cg * CG + cc
                        v = plsc.load_gather(
                            rows[p],
                            [rowidx[k], jnp.broadcast_to(jnp.int32(c), (L16,))],
                        )
                        tb[p][c, pl.ds(k * L16, L16)] = v + pb[cc]
            pltpu.async_copy(
                tb[p], out_hbm.at[l, :, pl.ds(b0, BBLK)], osem[p]
            )

        issue_gather(0, 0)

        def loop_body(j, carry):
            a = 2 * j
            @pl.when(j > 0)
            def _():
                wait_writeback(1)
            issue_gather(a + 1, 1)
            wait_gather(0)
            transpose_add_flush(a, 0)
            @pl.when(j < L // 2 - 1)
            def _():
                wait_writeback(0)
                issue_gather(a + 2, 0)
            wait_gather(1)
            transpose_add_flush(a + 1, 1)
            return carry

        lax.fori_loop(0, L // 2, loop_body, 0)
        wait_writeback(0)
        wait_writeback(1)

    return sc_kernel


def kernel(inputs, word_table, pos_table):
    B, L = inputs.shape
    V, D = word_table.shape
    idx_t = jnp.transpose(inputs).astype(jnp.int32)
    wpad = jnp.concatenate(
        [word_table, jnp.zeros((V, PADW - D), jnp.float32)], axis=1
    )
    ppad = jnp.concatenate(
        [pos_table, jnp.zeros((L, PADW - D), jnp.float32)], axis=1
    )
    sc = _make_sc_kernel(B, L, D, V)
    out_t = sc(idx_t, wpad, ppad)
    return jnp.transpose(out_t, (2, 0, 1))


# speedup vs baseline: 1.7985x; 1.7985x over previous
import functools

import jax
import jax.numpy as jnp
from jax import lax
from jax.experimental import pallas as pl
from jax.experimental.pallas import tpu as pltpu
from jax.experimental.pallas import tpu_sc as plsc

L16 = 16
PADW = 128


def _make_sc_kernel(B, L, D, V):
    info = plsc.get_sparse_core_info()
    NC, NS = info.num_cores, info.num_subcores
    NW = NC * NS
    BBLK = B // NW
    assert BBLK == 128 and L % 2 == 0 and D % L16 == 0
    NBV = BBLK // L16
    CG = 8

    mesh = plsc.VectorSubcoreMesh(core_axis_name="c", subcore_axis_name="s")

    @functools.partial(
        pl.kernel,
        mesh=mesh,
        compiler_params=pltpu.CompilerParams(
            use_tc_tiling_on_sc=True, needs_layout_passes=False
        ),
        out_type=jax.ShapeDtypeStruct((L, D, B), jnp.float32),
        scratch_types=[
            pltpu.VMEM((L, PADW), jnp.float32),
            pltpu.VMEM((L, BBLK), jnp.int32),
            pltpu.VMEM((BBLK, PADW), jnp.float32),
            pltpu.VMEM((BBLK, PADW), jnp.float32),
            pltpu.VMEM((D, BBLK), jnp.float32),
            pltpu.VMEM((D, BBLK), jnp.float32),
            pltpu.SemaphoreType.DMA,
            pltpu.SemaphoreType.DMA,
            pltpu.SemaphoreType.DMA,
            pltpu.SemaphoreType.DMA,
        ],
    )
    def sc_kernel(idx_hbm, word_hbm, pos_hbm, out_hbm,
                  pos_v, idx_v, rows0, rows1, t0, t1,
                  gsem0, gsem1, osem0, osem1):
        rows = (rows0, rows1)
        tb = (t0, t1)
        gsem = (gsem0, gsem1)
        osem = (osem0, osem1)
        wid = lax.axis_index("s") * NC + lax.axis_index("c")
        b0 = wid * BBLK
        pltpu.sync_copy(pos_hbm, pos_v)
        pltpu.sync_copy(idx_hbm.at[:, pl.ds(b0, BBLK)], idx_v)

        def issue_gather(l, p):
            pltpu.async_copy(word_hbm.at[idx_v.at[l]], rows[p], gsem[p])

        def wait_gather(p):
            pltpu.make_async_copy(
                word_hbm.at[pl.ds(0, BBLK)], rows[p], gsem[p]
            ).wait()

        def wait_writeback(p):
            pltpu.make_async_copy(
                tb[p], out_hbm.at[0, :, pl.ds(0, BBLK)], osem[p]
            ).wait()

        def transpose_add_flush(l, p):
            rowidx = [
                jnp.broadcast_to(jnp.int32(k * L16), (L16,))
                + lax.iota(jnp.int32, L16)
                for k in range(NBV)
            ]
            cvec = [jnp.broadcast_to(jnp.int32(c), (L16,)) for c in range(D)]
            pidx = jnp.broadcast_to(l, (L16,))
            for cg in range(D // CG):
                cs = [cg * CG + cc for cc in range(CG)]
                pb = [plsc.load_gather(pos_v, [pidx, cvec[c]]) for c in cs]
                for k in range(NBV):
                    vs = [
                        plsc.load_gather(rows[p], [rowidx[k], cvec[c]])
                        for c in cs
                    ]
                    for cc, c in enumerate(cs):
                        tb[p][c, pl.ds(k * L16, L16)] = vs[cc] + pb[cc]
            pltpu.async_copy(
                tb[p], out_hbm.at[l, :, pl.ds(b0, BBLK)], osem[p]
            )

        issue_gather(0, 0)

        def loop_body(j, carry):
            a = 2 * j
            @pl.when(j > 0)
            def _():
                wait_writeback(1)
            issue_gather(a + 1, 1)
            wait_gather(0)
            transpose_add_flush(a, 0)
            @pl.when(j < L // 2 - 1)
            def _():
                wait_writeback(0)
                issue_gather(a + 2, 0)
            wait_gather(1)
            transpose_add_flush(a + 1, 1)
            return carry

        lax.fori_loop(0, L // 2, loop_body, 0)
        wait_writeback(0)
        wait_writeback(1)

    return sc_kernel


def kernel(inputs, word_table, pos_table):
    B, L = inputs.shape
    V, D = word_table.shape
    idx_t = jnp.transpose(inputs).astype(jnp.int32)
    wpad = jnp.concatenate(
        [word_table, jnp.zeros((V, PADW - D), jnp.float32)], axis=1
    )
    ppad = jnp.concatenate(
        [pos_table, jnp.zeros((L, PADW - D), jnp.float32)], axis=1
    )
    sc = _make_sc_kernel(B, L, D, V)
    out_t = sc(idx_t, wpad, ppad)
    return jnp.transpose(out_t, (2, 0, 1))
